# baseline (speedup 1.0000x reference)
import functools

import jax
import jax.numpy as jnp
from jax import lax
from jax.experimental import pallas as pl
from jax.experimental.pallas import tpu as pltpu
from jax.experimental.pallas import tpu_sc as plsc

OUTPUT_DIMENSION = 64
SCALE = 8.0

NUM_CORES = 2
NUM_SUBCORES = 16
NUM_WORKERS = NUM_CORES * NUM_SUBCORES

B_TOTAL = 4096 * 200
B_PER_WORKER = B_TOTAL // NUM_WORKERS
CHUNK = 128
NUM_CHUNKS = B_PER_WORKER // CHUNK

_mesh = plsc.VectorSubcoreMesh(core_axis_name="c", subcore_axis_name="s")


@functools.partial(
    pl.kernel,
    mesh=_mesh,
    compiler_params=pltpu.CompilerParams(use_tc_tiling_on_sc=False),
    out_type=jax.ShapeDtypeStruct((B_TOTAL, OUTPUT_DIMENSION), jnp.float32),
    scratch_types=[
        pltpu.VMEM((NUM_CHUNKS, CHUNK), jnp.int32),
        pltpu.VMEM((CHUNK, OUTPUT_DIMENSION), jnp.float32),
        pltpu.SemaphoreType.DMA,
    ],
)
def _emb_lookup(idx_hbm, table_hbm, out_hbm, idx_v, rows_v, sem):
    wid = lax.axis_index("s") * NUM_CORES + lax.axis_index("c")
    base = wid * B_PER_WORKER
    pltpu.sync_copy(idx_hbm.at[pl.ds(wid * NUM_CHUNKS, NUM_CHUNKS)], idx_v)

    def chunk_body(ci, carry):
        off = base + ci * CHUNK
        pltpu.async_copy(table_hbm.at[idx_v.at[ci]], rows_v, sem).wait()

        def add_body(i, c):
            r = i // 4
            col = (i % 4) * 16
            rows_v[r, pl.ds(col, 16)] = rows_v[r, pl.ds(col, 16)] + SCALE
            return c

        lax.fori_loop(0, CHUNK * 4, add_body, 0)
        pltpu.sync_copy(rows_v, out_hbm.at[pl.ds(off, CHUNK)])
        return carry

    lax.fori_loop(0, NUM_CHUNKS, chunk_body, 0)


def kernel(x, embedding_table):
    idx = x.reshape(NUM_WORKERS * NUM_CHUNKS, CHUNK).astype(jnp.int32)
    out = _emb_lookup(idx, embedding_table)
    return out.reshape(x.shape + (OUTPUT_DIMENSION,))


# speedup vs baseline: 1.5835x; 1.5835x over previous
import functools

import jax
import jax.numpy as jnp
from jax import lax
from jax.experimental import pallas as pl
from jax.experimental.pallas import tpu as pltpu
from jax.experimental.pallas import tpu_sc as plsc

OUTPUT_DIMENSION = 64
SCALE = 8.0

NUM_CORES = 2
NUM_SUBCORES = 16
NUM_WORKERS = NUM_CORES * NUM_SUBCORES

B_TOTAL = 4096 * 200
B_PER_WORKER = B_TOTAL // NUM_WORKERS
IDX_ROW = 128
IDX_ROWS_PER_WORKER = B_PER_WORKER // IDX_ROW
CHUNK = 256
SUB = CHUNK // IDX_ROW
NUM_CHUNKS = B_PER_WORKER // CHUNK
NBUF = 4
K_ITERS = NUM_CHUNKS // NBUF

_mesh = plsc.VectorSubcoreMesh(core_axis_name="c", subcore_axis_name="s")


@functools.partial(
    pl.kernel,
    mesh=_mesh,
    compiler_params=pltpu.CompilerParams(use_tc_tiling_on_sc=False),
    out_type=jax.ShapeDtypeStruct((B_TOTAL, OUTPUT_DIMENSION), jnp.float32),
    scratch_types=[
        pltpu.VMEM((IDX_ROWS_PER_WORKER, IDX_ROW), jnp.int32),
        pltpu.VMEM((CHUNK, OUTPUT_DIMENSION), jnp.float32),
        pltpu.VMEM((CHUNK, OUTPUT_DIMENSION), jnp.float32),
        pltpu.VMEM((CHUNK, OUTPUT_DIMENSION), jnp.float32),
        pltpu.VMEM((CHUNK, OUTPUT_DIMENSION), jnp.float32),
        pltpu.SemaphoreType.DMA,
        pltpu.SemaphoreType.DMA,
        pltpu.SemaphoreType.DMA,
        pltpu.SemaphoreType.DMA,
        pltpu.SemaphoreType.DMA,
        pltpu.SemaphoreType.DMA,
        pltpu.SemaphoreType.DMA,
        pltpu.SemaphoreType.DMA,
    ],
)
def _emb_lookup(idx_hbm, table_hbm, out_hbm, idx_v,
                buf0, buf1, buf2, buf3,
                semg0, semg1, semg2, semg3,
                semw0, semw1, semw2, semw3):
    bufs = (buf0, buf1, buf2, buf3)
    semg = (semg0, semg1, semg2, semg3)
    semw = (semw0, semw1, semw2, semw3)

    wid = lax.axis_index("s") * NUM_CORES + lax.axis_index("c")
    base = wid * B_PER_WORKER
    pltpu.sync_copy(
        idx_hbm.at[pl.ds(wid * IDX_ROWS_PER_WORKER, IDX_ROWS_PER_WORKER)],
        idx_v,
    )

    def fire_gather(c, b):
        for j in range(SUB):
            pltpu.async_copy(
                table_hbm.at[idx_v.at[SUB * c + j]],
                bufs[b].at[pl.ds(j * IDX_ROW, IDX_ROW)],
                semg[b],
            )

    def drain_gather(b):
        pltpu.make_async_copy(out_hbm.at[pl.ds(0, CHUNK)], bufs[b], semg[b]).wait()

    def fire_writeback(c, b):
        pltpu.async_copy(bufs[b], out_hbm.at[pl.ds(base + c * CHUNK, CHUNK)], semw[b])

    def drain_writeback(b):
        pltpu.make_async_copy(bufs[b], out_hbm.at[pl.ds(0, CHUNK)], semw[b]).wait()

    def add_scale(b):
        buf = bufs[b]

        def body(r8, carry):
            for dr in range(8):
                for j in range(OUTPUT_DIMENSION // 16):
                    r = r8 * 8 + dr
                    col = j * 16
                    buf[r, pl.ds(col, 16)] = buf[r, pl.ds(col, 16)] + SCALE
            return carry

        lax.fori_loop(0, CHUNK // 8, body, 0)

    fire_gather(0, 0)

    def k_body(k, carry):
        for u in range(NBUF):
            c = NBUF * k + u
            nxt = (u + 1) % NBUF
            if u < NBUF - 1:
                @pl.when(k > 0)
                def _():
                    drain_writeback(nxt)
                fire_gather(c + 1, nxt)
            else:
                drain_writeback(0)

                @pl.when(k < K_ITERS - 1)
                def _():
                    fire_gather(c + 1, 0)
            drain_gather(u)
            add_scale(u)
            fire_writeback(c, u)
        return carry

    lax.fori_loop(0, K_ITERS, k_body, 0)
    drain_writeback(1)
    drain_writeback(2)
    drain_writeback(3)


def kernel(x, embedding_table):
    idx = x.reshape(
        NUM_WORKERS * IDX_ROWS_PER_WORKER, IDX_ROW).astype(jnp.int32)
    out = _emb_lookup(idx, embedding_table)
    return out.reshape(x.shape + (OUTPUT_DIMENSION,))
